# baseline (speedup 1.0000x reference)
import jax
import jax.numpy as jnp
from jax import lax
from jax.experimental import pallas as pl
from jax.experimental.pallas import tpu as pltpu
from jax.experimental.pallas import tpu_sc as plsc

FIELD_DIMS_N = 4000
NUM_FIELDS = 26
EMBED_DIM = 64
MAX_K = 512
M = 4
N_ACTION = 3
BATCH = 4096
SUM_FIELDS = NUM_FIELDS * FIELD_DIMS_N
PLEN = EMBED_DIM // M
TOK = BATCH * NUM_FIELDS

NC = 2
NS = 16
NW = NC * NS
PER_W = TOK // NW
T = 128
NBLK = PER_W // T
L = 16
NCB = N_ACTION * M
TABW = MAX_K * EMBED_DIM


def _body(xt_hbm, ap_hbm, cbi_hbm, cbk_hbm, out_hbm,
          gidblk, ap_v, tab, cbgidx, codes, outb,
          sem_c0, sem_c1, sem_o0, sem_o1):
    wid = lax.axis_index("s") * NC + lax.axis_index("c")
    base = wid * PER_W
    sem_c = (sem_c0, sem_c1)
    sem_o = (sem_o0, sem_o1)
    pltpu.sync_copy(ap_hbm, ap_v)

    def blk_field(blk):
        return (base + blk * T) // BATCH

    def stage_table(f):
        pltpu.sync_copy(cbk_hbm.at[pl.ds(f * TABW, TABW)], tab)

    def stage_codes(blk, par):
        t0 = base + blk * T
        f = blk_field(blk)
        pltpu.sync_copy(xt_hbm.at[pl.ds(t0, T)], gidblk)
        fo = f * FIELD_DIMS_N
        for g in range(T // L):
            gidv = gidblk[pl.ds(g * L, L)] + fo
            for c in range(NCB):
                cbgidx[par][c, pl.ds(g * L, L)] = gidv + (c * SUM_FIELDS)
        for c in range(NCB):
            pltpu.async_copy(cbi_hbm.at[cbgidx[par].at[c]],
                             codes[par].at[c], sem_c[par])

    def wait_codes(par):
        for c in range(NCB):
            pltpu.make_async_copy(cbi_hbm.at[cbgidx[par].at[c]],
                                  codes[par].at[c], sem_c[par]).wait()

    def combine(blk, par):
        t0 = base + blk * T
        f = blk_field(blk)
        b0 = t0 - f * BATCH
        f3 = f * N_ACTION
        ap = [ap_v[f3 + j, :] for j in range(N_ACTION)]

        def grp_body(g):
            cb = [codes[par][c, pl.ds(g * L, L)] * EMBED_DIM
                  for c in range(NCB)]
            for e in range(EMBED_DIM):
                i = e // PLEN
                vals = [plsc.load_gather(tab, [cb[j * M + i] + e])
                        for j in range(N_ACTION)]
                acc = (ap[0] * vals[0] + ap[1] * vals[1] + ap[2] * vals[2])
                outb[par][e, pl.ds(g * L, L)] = acc

        plsc.parallel_loop(0, T // L, 1)(grp_body)
        pltpu.async_copy(outb[par], out_hbm.at[f, :, pl.ds(b0, T)],
                         sem_o[par])

    def wait_out(blk, par):
        t0 = base + blk * T
        f = blk_field(blk)
        b0 = t0 - f * BATCH
        pltpu.make_async_copy(outb[par], out_hbm.at[f, :, pl.ds(b0, T)],
                              sem_o[par]).wait()

    stage_table(blk_field(0))
    stage_codes(0, 0)

    def loop_body(k, f_prev):
        fp = f_prev
        for par in (0, 1):
            b = 2 * k + par
            stage_codes(b + 1, 1 - par)
            f_b = blk_field(b)

            @pl.when(f_b != fp)
            def _():
                stage_table(f_b)
            fp = f_b

            @pl.when(k >= 1)
            def _():
                wait_out(b - 2, par)
            wait_codes(par)
            combine(b, par)
        return fp

    f_prev = lax.fori_loop(0, NBLK // 2 - 1, loop_body, blk_field(0))

    b = NBLK - 2
    stage_codes(b + 1, 1)
    for par, bb in ((0, b), (1, b + 1)):
        f_b = blk_field(bb)

        @pl.when(f_b != f_prev)
        def _():
            stage_table(f_b)
        f_prev = f_b
        wait_out(bb - 2, par)
        wait_codes(par)
        combine(bb, par)
    wait_out(b, 0)
    wait_out(b + 1, 1)


@jax.jit
def kernel(x, arch_prob, codebooks, cb_index):
    xt_flat = x.T.reshape(TOK)
    ap_splat = jnp.broadcast_to(
        arch_prob.reshape(NUM_FIELDS * N_ACTION, 1), (NUM_FIELDS * N_ACTION, L)
    )
    cbi_flat = cb_index.transpose(0, 2, 1).reshape(-1)
    cbk_flat = codebooks.reshape(-1)

    mesh = plsc.VectorSubcoreMesh(core_axis_name="c", subcore_axis_name="s")
    dbl = lambda sh, dt: [pltpu.VMEM(sh, dt), pltpu.VMEM(sh, dt)]
    run = pl.kernel(
        _body,
        out_type=jax.ShapeDtypeStruct((NUM_FIELDS, EMBED_DIM, BATCH),
                                      jnp.float32),
        mesh=mesh,
        compiler_params=pltpu.CompilerParams(use_tc_tiling_on_sc=False,
                                             needs_layout_passes=False),
        scratch_types=[
            pltpu.VMEM((T,), jnp.int32),
            pltpu.VMEM((NUM_FIELDS * N_ACTION, L), jnp.float32),
            pltpu.VMEM((TABW,), jnp.float32),
            dbl((NCB, T), jnp.int32),
            dbl((NCB, T), jnp.int32),
            dbl((EMBED_DIM, T), jnp.float32),
            pltpu.SemaphoreType.DMA,
            pltpu.SemaphoreType.DMA,
            pltpu.SemaphoreType.DMA,
            pltpu.SemaphoreType.DMA,
        ],
    )
    out = run(xt_flat, ap_splat, cbi_flat, cbk_flat)
    return out.transpose(2, 0, 1)


# speedup vs baseline: 1.8654x; 1.8654x over previous
import jax
import jax.numpy as jnp
from jax import lax
from jax.experimental import pallas as pl
from jax.experimental.pallas import tpu as pltpu
from jax.experimental.pallas import tpu_sc as plsc

FIELD_DIMS_N = 4000
NUM_FIELDS = 26
EMBED_DIM = 64
MAX_K = 512
M = 4
N_ACTION = 3
BATCH = 4096
SUM_FIELDS = NUM_FIELDS * FIELD_DIMS_N
PLEN = EMBED_DIM // M
TOK = BATCH * NUM_FIELDS

NC = 2
NS = 16
NW = NC * NS
PER_W = TOK // NW
T = 128
NBLK = PER_W // T
L = 16
NCB = N_ACTION * M


def _body(gid_hbm, ap_hbm, cbi_hbm, cbk_hbm, out_hbm,
          gidblk, ap_v, cbgidx, codes, cbidx, rows, outb,
          sem_c0, sem_c1, sem_r0, sem_r1, sem_o0, sem_o1):
    wid = lax.axis_index("s") * NC + lax.axis_index("c")
    base = wid * PER_W
    sem_c = (sem_c0, sem_c1)
    sem_r = (sem_r0, sem_r1)
    sem_o = (sem_o0, sem_o1)
    pltpu.sync_copy(ap_hbm, ap_v)
    iota = lax.iota(jnp.int32, L)

    def stage_codes(blk, par):
        t0 = base + blk * T
        pltpu.sync_copy(gid_hbm.at[pl.ds(t0, T)], gidblk)
        for g in range(T // L):
            gidv = gidblk[pl.ds(g * L, L)]
            for c in range(NCB):
                cbgidx[par][c, pl.ds(g * L, L)] = gidv + (c * SUM_FIELDS)
        for c in range(NCB):
            pltpu.async_copy(cbi_hbm.at[cbgidx[par].at[c]],
                             codes[par].at[c], sem_c[par])

    def wait_codes(par):
        for c in range(NCB):
            pltpu.make_async_copy(cbi_hbm.at[cbgidx[par].at[c]],
                                  codes[par].at[c], sem_c[par]).wait()

    def stage_rows(blk, par):
        t0 = base + blk * T
        for g in range(T // L):
            fb = lax.rem(iota + (t0 + g * L), NUM_FIELDS) * (MAX_K * M)
            for c in range(NCB):
                cv = codes[par][c, pl.ds(g * L, L)]
                cbidx[par][c, pl.ds(g * L, L)] = fb + cv * M + (c % M)
        for c in range(NCB):
            pltpu.async_copy(cbk_hbm.at[cbidx[par].at[c]],
                             rows[par].at[c], sem_r[par])

    def wait_rows(par):
        for c in range(NCB):
            pltpu.make_async_copy(cbk_hbm.at[cbidx[par].at[c]],
                                  rows[par].at[c], sem_r[par]).wait()

    def combine(blk, par):
        t0 = base + blk * T

        def tok_body(t, carry2):
            f3 = lax.rem(t0 + t, NUM_FIELDS) * N_ACTION
            ap0 = ap_v[f3, :]
            ap1 = ap_v[f3 + 1, :]
            ap2 = ap_v[f3 + 2, :]
            for i in range(M):
                acc = (ap0 * rows[par][i, t, :]
                       + ap1 * rows[par][M + i, t, :]
                       + ap2 * rows[par][2 * M + i, t, :])
                outb[par][t, pl.ds(i * PLEN, PLEN)] = acc
            return carry2

        plsc.parallel_loop(0, T, 1, unroll=16)(lambda t: tok_body(t, 0))
        pltpu.async_copy(outb[par], out_hbm.at[pl.ds(t0, T)], sem_o[par])

    def wait_out(blk, par):
        t0 = base + blk * T
        pltpu.make_async_copy(outb[par], out_hbm.at[pl.ds(t0, T)],
                              sem_o[par]).wait()

    stage_codes(0, 0)
    wait_codes(0)
    stage_rows(0, 0)
    stage_codes(1, 1)

    def loop_body(k, carry):
        for par in (0, 1):
            b = 2 * k + par
            wait_codes(1 - par)
            stage_rows(b + 1, 1 - par)
            stage_codes(b + 2, par)

            @pl.when(k >= 1)
            def _():
                wait_out(b - 2, par)
            wait_rows(par)
            combine(b, par)
        return carry

    lax.fori_loop(0, NBLK // 2 - 1, loop_body, 0)

    b = NBLK - 2
    wait_codes(1)
    stage_rows(b + 1, 1)
    wait_out(b - 2, 0)
    wait_rows(0)
    combine(b, 0)
    wait_out(b - 1, 1)
    wait_rows(1)
    combine(b + 1, 1)
    wait_out(b, 0)
    wait_out(b + 1, 1)


@jax.jit
def kernel(x, arch_prob, codebooks, cb_index):
    offsets = jnp.arange(NUM_FIELDS, dtype=jnp.int32) * FIELD_DIMS_N
    gid = (x + offsets[None, :]).reshape(TOK)
    ap_splat = jnp.broadcast_to(
        arch_prob.reshape(NUM_FIELDS * N_ACTION, 1), (NUM_FIELDS * N_ACTION, L)
    )
    cbi_flat = cb_index.transpose(0, 2, 1).reshape(-1)
    cbk = codebooks.reshape(NUM_FIELDS * MAX_K * M, PLEN)

    mesh = plsc.VectorSubcoreMesh(core_axis_name="c", subcore_axis_name="s")
    dbl = lambda sh, dt: [pltpu.VMEM(sh, dt), pltpu.VMEM(sh, dt)]
    run = pl.kernel(
        _body,
        out_type=jax.ShapeDtypeStruct((TOK, EMBED_DIM), jnp.float32),
        mesh=mesh,
        compiler_params=pltpu.CompilerParams(use_tc_tiling_on_sc=False),
        scratch_types=[
            pltpu.VMEM((T,), jnp.int32),
            pltpu.VMEM((NUM_FIELDS * N_ACTION, L), jnp.float32),
            dbl((NCB, T), jnp.int32),
            dbl((NCB, T), jnp.int32),
            dbl((NCB, T), jnp.int32),
            dbl((NCB, T, PLEN), jnp.float32),
            dbl((T, EMBED_DIM), jnp.float32),
            pltpu.SemaphoreType.DMA,
            pltpu.SemaphoreType.DMA,
            pltpu.SemaphoreType.DMA,
            pltpu.SemaphoreType.DMA,
            pltpu.SemaphoreType.DMA,
            pltpu.SemaphoreType.DMA,
        ],
    )
    out = run(gid, ap_splat, cbi_flat, cbk)
    return out.reshape(BATCH, NUM_FIELDS, EMBED_DIM)
